# baseline (speedup 1.0000x reference)
import jax
import jax.numpy as jnp
from jax import lax
from jax.experimental import pallas as pl
from jax.experimental.pallas import tpu as pltpu
from jax.experimental.pallas import tpu_sc as plsc

N = 10000
E = 320000
H = 128
D_EDGE = 16
ACCW = 144
NW = 32
EPW = E // NW
B = 40
CH = EPW // B
BCH = 25
NBL = CH // BCH
RPT = 624
DENR = 80

_f32 = jnp.float32


def _node_body(x_ref, wn_ref, bn_ref, a1_ref, a2_ref, o1_ref, o2_ref):
    h = jnp.dot(x_ref[...], wn_ref[...], preferred_element_type=_f32) + bn_ref[...]
    o1_ref[...] = jnp.dot(h, a1_ref[...], preferred_element_type=_f32)
    o2_ref[:, :H] = jnp.dot(h, a2_ref[...], preferred_element_type=_f32)
    o2_ref[:, H:] = h


def _node_tc(x, Wn, bn, A1, A2):
    blk = 400
    grid = N // blk
    return pl.pallas_call(
        _node_body,
        grid=(grid,),
        in_specs=[
            pl.BlockSpec((blk, H), lambda i: (i, 0)),
            pl.BlockSpec((H, H), lambda i: (0, 0)),
            pl.BlockSpec((1, H), lambda i: (0, 0)),
            pl.BlockSpec((H, H), lambda i: (0, 0)),
            pl.BlockSpec((H, H), lambda i: (0, 0)),
        ],
        out_specs=[
            pl.BlockSpec((blk, H), lambda i: (i, 0)),
            pl.BlockSpec((blk, 2 * H), lambda i: (i, 0)),
        ],
        out_shape=[
            jax.ShapeDtypeStruct((N, H), _f32),
            jax.ShapeDtypeStruct((N, 2 * H), _f32),
        ],
    )(x, Wn, bn, A1, A2)


def _edge_body(ea_ref, we_ref, be_ref, a3_ref, ba1_ref, o_ref):
    a3 = a3_ref[...]
    wc = jnp.dot(we_ref[...], a3, preferred_element_type=_f32)
    bc = jnp.dot(be_ref[...], a3, preferred_element_type=_f32) + ba1_ref[...]
    o_ref[...] = jnp.dot(ea_ref[...], wc, preferred_element_type=_f32) + bc


def _edge_tc(ea, We, be, A3, ba1):
    blk = 2000
    grid = E // blk
    return pl.pallas_call(
        _edge_body,
        grid=(grid,),
        in_specs=[
            pl.BlockSpec((blk, D_EDGE), lambda i: (i, 0)),
            pl.BlockSpec((D_EDGE, H), lambda i: (0, 0)),
            pl.BlockSpec((1, H), lambda i: (0, 0)),
            pl.BlockSpec((H, H), lambda i: (0, 0)),
            pl.BlockSpec((1, H), lambda i: (0, 0)),
        ],
        out_specs=pl.BlockSpec((blk, H), lambda i: (i, 0)),
        out_shape=jax.ShapeDtypeStruct((E, H), _f32),
    )(ea, We, be, A3, ba1)


def _sc_body(tsrc, tdst, ea3, src3, dst3, params, zeros, out_num, out_den,
             acc, den_sh, sidx1, didx1, Sv0, Sv1, Dv0, Dv1, E0, E1,
             outv0, outv1, svbuf, scat0, scat1, den_t, parv, idbuf,
             semA, semB, semS0, semS1):
    cid = lax.axis_index("c")
    sid = lax.axis_index("s")
    wid = cid * 16 + sid

    pltpu.sync_copy(params, parv)
    pltpu.sync_copy(zeros.at[pl.ds(0, DENR)], den_t)
    row0 = pl.multiple_of(sid * RPT, 8)
    tail = 16 * RPT
    pltpu.sync_copy(zeros.at[pl.ds(row0, RPT)], acc.at[pl.ds(row0, RPT)])

    @pl.when(sid == 15)
    def _zero_tail():
        pltpu.sync_copy(zeros.at[pl.ds(tail, N - tail)],
                        acc.at[pl.ds(tail, N - tail)])

    @pl.when(sid == 0)
    def _zero_den():
        pltpu.sync_copy(zeros.at[pl.ds(0, DENR)], den_sh)

    for k in range(DENR // 16):
        idbuf[pl.ds(16 * k, 16)] = lax.iota(jnp.int32, 16) + 16 * k

    w2 = [parv[pl.ds(16 * j, 16)] for j in range(8)]
    cvv = parv[pl.ds(H, 16)]
    lanes = lax.iota(jnp.int32, 16)
    perms = [(lanes + sh) & 15 for sh in (8, 4, 2, 1)]
    ilv = plsc.PackFormat.INTERLEAVED

    plsc.subcore_barrier()

    def issue(cl, Sb, Db, Eb, sem, gbase):
        o = pl.multiple_of(cl * B, 8)
        pltpu.async_copy(tsrc.at[sidx1.at[pl.ds(o, B)]], Sb, sem)
        pltpu.async_copy(tdst.at[didx1.at[pl.ds(o, B)]], Db, sem)
        pltpu.async_copy(ea3.at[gbase + cl], Eb, sem)

    def wait_buf(Sb, Db, Eb, sem):
        pltpu.make_async_copy(tsrc.at[pl.ds(0, B)], Sb, sem).wait()
        pltpu.make_async_copy(tdst.at[pl.ds(0, B)], Db, sem).wait()
        pltpu.make_async_copy(ea3.at[0], Eb, sem).wait()

    def compute(cl, gc, Sb, Db, Eb, outb, scatb, semS):
        @pl.when(gc >= 2)
        def _wait_prev_scatter():
            pltpu.make_async_copy(outb, acc.at[pl.ds(0, B)], semS).wait()

        def edge_body(e, carry):
            a0 = jnp.zeros((16,), _f32)
            a1 = jnp.zeros((16,), _f32)
            for g in range(4):
                da, db = plsc.unpack(
                    plsc.bitcast(Db[e, pl.ds(16 * g, 16)], jnp.bfloat16),
                    format=ilv)
                ea_, eb_ = plsc.unpack(
                    plsc.bitcast(Eb[pl.ds(e * 64 + 16 * g, 16)], jnp.bfloat16),
                    format=ilv)
                xa = Sb[e, pl.ds(32 * g, 16)] + da + ea_
                xb = Sb[e, pl.ds(32 * g + 16, 16)] + db + eb_
                ra = 1.0 / (jnp.exp(xa) + 1.0)
                rb = 1.0 / (jnp.exp(xb) + 1.0)
                a0 = a0 + w2[2 * g] * ra
                a1 = a1 + w2[2 * g + 1] * rb
            red = a0 + a1
            for p in perms:
                red = red + red.at[p].get(mode="promise_in_bounds")
            sv = jnp.exp(cvv - red)
            for g in range(4):
                ha, hb = plsc.unpack(
                    plsc.bitcast(Db[e, pl.ds(64 + 16 * g, 16)], jnp.bfloat16),
                    format=ilv)
                outb[e, pl.ds(32 * g, 16)] = sv * ha
                outb[e, pl.ds(32 * g + 16, 16)] = sv * hb
            plsc.store_scatter(svbuf, [jnp.full((16,), e, jnp.int32)], sv,
                               mask=lanes == 0)
            return carry

        lax.fori_loop(0, B, edge_body, 0, unroll=4)
        o = pl.multiple_of(cl * B, 8)
        srcv0 = sidx1[pl.ds(o, 16)]
        srcv1 = sidx1[pl.ds(o + 16, 16)]
        srcvt = sidx1[pl.ds(o + 24, 16)]
        scatb[pl.ds(0, 16)] = srcv0
        scatb[pl.ds(16, 16)] = srcv1
        scatb[pl.ds(24, 16)] = srcvt
        plsc.addupdate_scatter(
            den_t, [lax.shift_right_logical(srcv0, 7), srcv0 & 127],
            svbuf[pl.ds(0, 16)])
        plsc.addupdate_scatter(
            den_t, [lax.shift_right_logical(srcv1, 7), srcv1 & 127],
            svbuf[pl.ds(16, 16)])
        plsc.addupdate_scatter(
            den_t, [lax.shift_right_logical(srcvt, 7), srcvt & 127],
            svbuf[pl.ds(24, 16)], mask=lanes >= 8)
        pltpu.async_copy(outb, acc.at[scatb], semS, add=True)

    def block_fn(bi, carry):
        gbase = wid * CH + bi * BCH
        pltpu.sync_copy(src3.at[wid, bi], sidx1)
        pltpu.sync_copy(dst3.at[wid, bi], didx1)
        issue(0, Sv0, Dv0, E0, semA, gbase)

        def pair_fn(k, carry2):
            cl = k * 2
            issue(cl + 1, Sv1, Dv1, E1, semB, gbase)
            wait_buf(Sv0, Dv0, E0, semA)
            compute(cl, bi * BCH + cl, Sv0, Dv0, E0, outv0, scat0, semS0)
            issue(cl + 2, Sv0, Dv0, E0, semA, gbase)
            wait_buf(Sv1, Dv1, E1, semB)
            compute(cl + 1, bi * BCH + cl + 1, Sv1, Dv1, E1, outv1, scat1,
                    semS1)
            return carry2

        lax.fori_loop(0, (BCH - 1) // 2, pair_fn, 0)
        wait_buf(Sv0, Dv0, E0, semA)
        compute(BCH - 1, bi * BCH + BCH - 1, Sv0, Dv0, E0, outv0, scat0, semS0)
        return carry

    lax.fori_loop(0, NBL, block_fn, 0)

    pltpu.make_async_copy(outv0, acc.at[pl.ds(0, B)], semS0).wait()
    pltpu.make_async_copy(outv1, acc.at[pl.ds(0, B)], semS1).wait()

    pltpu.sync_copy(den_t, den_sh.at[idbuf], add=True)
    plsc.subcore_barrier()

    pltpu.sync_copy(acc.at[pl.ds(row0, RPT)], out_num.at[cid, pl.ds(row0, RPT)])

    @pl.when(sid == 15)
    def _write_tail():
        pltpu.sync_copy(acc.at[pl.ds(tail, N - tail)],
                        out_num.at[cid, pl.ds(tail, N - tail)])

    @pl.when(sid == 0)
    def _write_den():
        pltpu.sync_copy(den_sh, out_den.at[cid])


def _sc_edge(table_src, table_dst, ea3, src, dst, params, zeros):
    src3 = src.reshape(NW, NBL, BCH * B)
    dst3 = dst.reshape(NW, NBL, BCH * B)
    mesh = plsc.VectorSubcoreMesh(core_axis_name="c", subcore_axis_name="s")
    call = pl.kernel(
        _sc_body,
        out_type=(
            jax.ShapeDtypeStruct((2, N, H), _f32),
            jax.ShapeDtypeStruct((2, DENR, H), _f32),
        ),
        mesh=mesh,
        scratch_types=[
            pltpu.VMEM_SHARED((N, H), _f32),
            pltpu.VMEM_SHARED((DENR, H), _f32),
            pltpu.VMEM((BCH * B,), jnp.int32),
            pltpu.VMEM((BCH * B,), jnp.int32),
            pltpu.VMEM((B, H), _f32),
            pltpu.VMEM((B, H), _f32),
            pltpu.VMEM((B, H), jnp.int32),
            pltpu.VMEM((B, H), jnp.int32),
            pltpu.VMEM((B * H // 2,), jnp.int32),
            pltpu.VMEM((B * H // 2,), jnp.int32),
            pltpu.VMEM((B, H), _f32),
            pltpu.VMEM((B, H), _f32),
            pltpu.VMEM((B,), _f32),
            pltpu.VMEM((B,), jnp.int32),
            pltpu.VMEM((B,), jnp.int32),
            pltpu.VMEM((DENR, H), _f32),
            pltpu.VMEM((ACCW,), _f32),
            pltpu.VMEM((DENR,), jnp.int32),
            pltpu.SemaphoreType.DMA,
            pltpu.SemaphoreType.DMA,
            pltpu.SemaphoreType.DMA,
            pltpu.SemaphoreType.DMA,
        ],
        compiler_params=pltpu.CompilerParams(needs_layout_passes=False),
    )
    return call(table_src, table_dst, ea3, src3, dst3, params, zeros)


def _final_body(n0_ref, n1_ref, d0_ref, d1_ref, wo_ref, bo_ref, g_ref, b_ref,
                o_ref):
    num = n0_ref[...] + n1_ref[...]
    den = d0_ref[...] + d1_ref[...]
    agg = jnp.where(den > 0.0, num / den, 0.0)
    o = jnp.dot(agg, wo_ref[...], preferred_element_type=_f32) + bo_ref[...]
    mu = jnp.mean(o, axis=-1, keepdims=True)
    var = jnp.mean((o - mu) ** 2, axis=-1, keepdims=True)
    o = g_ref[...] * (o - mu) * lax.rsqrt(var + 1e-5) + b_ref[...]
    o_ref[...] = jnp.maximum(o, 0.0)


def _final_tc(n0, n1, d0, d1, Wo, bo, g, b):
    blk = 400
    grid = N // blk
    return pl.pallas_call(
        _final_body,
        grid=(grid,),
        in_specs=[
            pl.BlockSpec((blk, H), lambda i: (i, 0)),
            pl.BlockSpec((blk, H), lambda i: (i, 0)),
            pl.BlockSpec((blk, 1), lambda i: (i, 0)),
            pl.BlockSpec((blk, 1), lambda i: (i, 0)),
            pl.BlockSpec((H, H), lambda i: (0, 0)),
            pl.BlockSpec((1, H), lambda i: (0, 0)),
            pl.BlockSpec((1, H), lambda i: (0, 0)),
            pl.BlockSpec((1, H), lambda i: (0, 0)),
        ],
        out_specs=pl.BlockSpec((blk, H), lambda i: (i, 0)),
        out_shape=jax.ShapeDtypeStruct((N, H), _f32),
    )(n0, n1, d0, d1, Wo, bo, g, b)


def kernel(node_features, edge_index, edge_attr, Wn, bn, We, be,
           Wa1, ba1, Wa2, ba2, Wo, bo, ln_g, ln_b):
    tl = jnp.float32(2.0)
    A1 = Wa1[:H] * tl
    A2 = Wa1[H:2 * H] * tl
    A3 = Wa1[2 * H:] * tl
    table_src, table_dst = _node_tc(node_features, Wn, bn.reshape(1, H), A1, A2)
    ea3 = _edge_tc(edge_attr, We, be.reshape(1, H), A3, (ba1 * tl).reshape(1, H))
    table_dst = lax.bitcast_convert_type(
        table_dst.astype(jnp.bfloat16)
        .reshape(N, 2, 4, 2, 16).transpose(0, 1, 2, 4, 3)
        .reshape(N, H, 2), jnp.int32)
    ea3 = lax.bitcast_convert_type(
        ea3.astype(jnp.bfloat16)
        .reshape(E, 4, 2, 16).transpose(0, 1, 3, 2)
        .reshape(E, H // 2, 2), jnp.int32).reshape(E // B, B * H // 2)
    cv = jnp.sum(Wa2[:, 0]) + ba2[0]
    params = jnp.concatenate([Wa2[:, 0] * 2.0,
                              jnp.full((ACCW - H,), cv, _f32)])
    zeros = jnp.zeros((N, H), _f32)
    num, den = _sc_edge(table_src, table_dst, ea3,
                        edge_index[0], edge_index[1], params, zeros)
    d0 = den[0].reshape(DENR * H)[:N].reshape(N, 1)
    d1 = den[1].reshape(DENR * H)[:N].reshape(N, 1)
    return _final_tc(num[0], num[1], d0, d1, Wo, bo.reshape(1, H),
                     ln_g.reshape(1, H), ln_b.reshape(1, H))


# speedup vs baseline: 1.1919x; 1.1919x over previous
import jax
import jax.numpy as jnp
from jax import lax
from jax.experimental import pallas as pl
from jax.experimental.pallas import tpu as pltpu
from jax.experimental.pallas import tpu_sc as plsc

N = 10000
E = 320000
H = 128
D_EDGE = 16
ACCW = 144
NW = 32
EPW = E // NW
B = 16
CH = EPW // B
BCH = 125
NBL = CH // BCH
RPT = 624
DENR = 80

_f32 = jnp.float32


def _node_body(x_ref, wn_ref, bn_ref, a1_ref, a2_ref, o1_ref, o2_ref):
    h = jnp.dot(x_ref[...], wn_ref[...], preferred_element_type=_f32) + bn_ref[...]
    o1_ref[...] = jnp.dot(h, a1_ref[...], preferred_element_type=_f32)
    o2_ref[:, :H] = jnp.dot(h, a2_ref[...], preferred_element_type=_f32)
    o2_ref[:, H:] = h


def _node_tc(x, Wn, bn, A1, A2):
    blk = 400
    grid = N // blk
    return pl.pallas_call(
        _node_body,
        grid=(grid,),
        in_specs=[
            pl.BlockSpec((blk, H), lambda i: (i, 0)),
            pl.BlockSpec((H, H), lambda i: (0, 0)),
            pl.BlockSpec((1, H), lambda i: (0, 0)),
            pl.BlockSpec((H, H), lambda i: (0, 0)),
            pl.BlockSpec((H, H), lambda i: (0, 0)),
        ],
        out_specs=[
            pl.BlockSpec((blk, H), lambda i: (i, 0)),
            pl.BlockSpec((blk, 2 * H), lambda i: (i, 0)),
        ],
        out_shape=[
            jax.ShapeDtypeStruct((N, H), _f32),
            jax.ShapeDtypeStruct((N, 2 * H), _f32),
        ],
    )(x, Wn, bn, A1, A2)


def _edge_body(ea_ref, we_ref, be_ref, a3_ref, ba1_ref, o_ref):
    a3 = a3_ref[...]
    wc = jnp.dot(we_ref[...], a3, preferred_element_type=_f32)
    bc = jnp.dot(be_ref[...], a3, preferred_element_type=_f32) + ba1_ref[...]
    o_ref[...] = jnp.dot(ea_ref[...], wc, preferred_element_type=_f32) + bc


def _edge_tc(ea, We, be, A3, ba1):
    blk = 2000
    grid = E // blk
    return pl.pallas_call(
        _edge_body,
        grid=(grid,),
        in_specs=[
            pl.BlockSpec((blk, D_EDGE), lambda i: (i, 0)),
            pl.BlockSpec((D_EDGE, H), lambda i: (0, 0)),
            pl.BlockSpec((1, H), lambda i: (0, 0)),
            pl.BlockSpec((H, H), lambda i: (0, 0)),
            pl.BlockSpec((1, H), lambda i: (0, 0)),
        ],
        out_specs=pl.BlockSpec((blk, H), lambda i: (i, 0)),
        out_shape=jax.ShapeDtypeStruct((E, H), _f32),
    )(ea, We, be, A3, ba1)


def _sc_body(tsrc, tdst, ea3, src3, dst3, params, zeros, out_num, out_den,
             acc, den_sh, sidx1, didx1, Sv0, Sv1, Dv0, Dv1, E0, E1,
             outv0, outv1, svbuf, scat0, scat1, den_t, parv, idbuf,
             semA, semB, semS0, semS1):
    cid = lax.axis_index("c")
    sid = lax.axis_index("s")
    wid = cid * 16 + sid

    pltpu.sync_copy(params, parv)
    pltpu.sync_copy(zeros.at[pl.ds(0, DENR)], den_t)
    row0 = pl.multiple_of(sid * RPT, 8)
    tail = 16 * RPT
    pltpu.sync_copy(zeros.at[pl.ds(row0, RPT)], acc.at[pl.ds(row0, RPT)])

    @pl.when(sid == 15)
    def _zero_tail():
        pltpu.sync_copy(zeros.at[pl.ds(tail, N - tail)],
                        acc.at[pl.ds(tail, N - tail)])

    @pl.when(sid == 0)
    def _zero_den():
        pltpu.sync_copy(zeros.at[pl.ds(0, DENR)], den_sh)

    for k in range(DENR // 16):
        idbuf[pl.ds(16 * k, 16)] = lax.iota(jnp.int32, 16) + 16 * k

    w2 = [parv[pl.ds(16 * j, 16)] for j in range(8)]
    cvv = parv[pl.ds(H, 16)]
    lanes = lax.iota(jnp.int32, 16)
    perms = [(lanes + sh) & 15 for sh in (8, 4, 2, 1)]
    ilv = plsc.PackFormat.INTERLEAVED

    plsc.subcore_barrier()

    def issue(cl, Sb, Db, Eb, sem, gbase):
        o = pl.multiple_of(cl * B, 8)
        pltpu.async_copy(tsrc.at[sidx1.at[pl.ds(o, B)]], Sb, sem)
        pltpu.async_copy(tdst.at[didx1.at[pl.ds(o, B)]], Db, sem)
        pltpu.async_copy(ea3.at[pl.ds((gbase + cl) * B, B)], Eb, sem)

    def wait_buf(Sb, Db, Eb, sem):
        pltpu.make_async_copy(tsrc.at[pl.ds(0, B)], Sb, sem).wait()
        pltpu.make_async_copy(tdst.at[pl.ds(0, B)], Db, sem).wait()
        pltpu.make_async_copy(ea3.at[pl.ds(0, B)], Eb, sem).wait()

    def compute(cl, gc, Sb, Db, Eb, outb, scatb, semS):
        @pl.when(gc >= 2)
        def _wait_prev_scatter():
            pltpu.make_async_copy(outb, acc.at[pl.ds(0, B)], semS).wait()

        def edge_body(e, carry):
            a0 = jnp.zeros((16,), _f32)
            a1 = jnp.zeros((16,), _f32)
            for j in range(8):
                sl = pl.ds(16 * j, 16)
                x2 = Sb[e, sl] + Db[e, sl] + Eb[e, sl]
                r = 1.0 / (jnp.exp(x2) + 1.0)
                if j % 2 == 0:
                    a0 = a0 + w2[j] * r
                else:
                    a1 = a1 + w2[j] * r
            red = a0 + a1
            for p in perms:
                red = red + red.at[p].get(mode="promise_in_bounds")
            sv = jnp.exp(cvv - red)
            for j in range(8):
                outb[e, pl.ds(16 * j, 16)] = sv * Db[e, pl.ds(H + 16 * j, 16)]
            plsc.store_scatter(svbuf, [lanes], sv, mask=lanes == e)
            return carry

        lax.fori_loop(0, B, edge_body, 0, unroll=8)
        srcv = sidx1[pl.ds(pl.multiple_of(cl * B, 8), B)]
        scatb[...] = srcv
        sval = svbuf[...]
        plsc.addupdate_scatter(
            den_t, [lax.shift_right_logical(srcv, 7), srcv & 127], sval)
        pltpu.async_copy(outb, acc.at[scatb], semS, add=True)

    def block_fn(bi, carry):
        gbase = wid * CH + bi * BCH
        pltpu.sync_copy(src3.at[wid, bi], sidx1)
        pltpu.sync_copy(dst3.at[wid, bi], didx1)
        issue(0, Sv0, Dv0, E0, semA, gbase)

        def pair_fn(k, carry2):
            cl = k * 2
            issue(cl + 1, Sv1, Dv1, E1, semB, gbase)
            wait_buf(Sv0, Dv0, E0, semA)
            compute(cl, bi * BCH + cl, Sv0, Dv0, E0, outv0, scat0, semS0)
            issue(cl + 2, Sv0, Dv0, E0, semA, gbase)
            wait_buf(Sv1, Dv1, E1, semB)
            compute(cl + 1, bi * BCH + cl + 1, Sv1, Dv1, E1, outv1, scat1,
                    semS1)
            return carry2

        lax.fori_loop(0, (BCH - 1) // 2, pair_fn, 0)
        wait_buf(Sv0, Dv0, E0, semA)
        compute(BCH - 1, bi * BCH + BCH - 1, Sv0, Dv0, E0, outv0, scat0, semS0)
        return carry

    lax.fori_loop(0, NBL, block_fn, 0)

    pltpu.make_async_copy(outv0, acc.at[pl.ds(0, B)], semS0).wait()
    pltpu.make_async_copy(outv1, acc.at[pl.ds(0, B)], semS1).wait()

    pltpu.sync_copy(den_t, den_sh.at[idbuf], add=True)
    plsc.subcore_barrier()

    pltpu.sync_copy(acc.at[pl.ds(row0, RPT)], out_num.at[cid, pl.ds(row0, RPT)])

    @pl.when(sid == 15)
    def _write_tail():
        pltpu.sync_copy(acc.at[pl.ds(tail, N - tail)],
                        out_num.at[cid, pl.ds(tail, N - tail)])

    @pl.when(sid == 0)
    def _write_den():
        pltpu.sync_copy(den_sh, out_den.at[cid])


def _sc_edge(table_src, table_dst, ea3, src, dst, params, zeros):
    src3 = src.reshape(NW, NBL, BCH * B)
    dst3 = dst.reshape(NW, NBL, BCH * B)
    mesh = plsc.VectorSubcoreMesh(core_axis_name="c", subcore_axis_name="s")
    call = pl.kernel(
        _sc_body,
        out_type=(
            jax.ShapeDtypeStruct((2, N, H), _f32),
            jax.ShapeDtypeStruct((2, DENR, H), _f32),
        ),
        mesh=mesh,
        scratch_types=[
            pltpu.VMEM_SHARED((N, H), _f32),
            pltpu.VMEM_SHARED((DENR, H), _f32),
            pltpu.VMEM((BCH * B,), jnp.int32),
            pltpu.VMEM((BCH * B,), jnp.int32),
            pltpu.VMEM((B, H), _f32),
            pltpu.VMEM((B, H), _f32),
            pltpu.VMEM((B, 2 * H), _f32),
            pltpu.VMEM((B, 2 * H), _f32),
            pltpu.VMEM((B, H), _f32),
            pltpu.VMEM((B, H), _f32),
            pltpu.VMEM((B, H), _f32),
            pltpu.VMEM((B, H), _f32),
            pltpu.VMEM((16,), _f32),
            pltpu.VMEM((16,), jnp.int32),
            pltpu.VMEM((16,), jnp.int32),
            pltpu.VMEM((DENR, H), _f32),
            pltpu.VMEM((ACCW,), _f32),
            pltpu.VMEM((DENR,), jnp.int32),
            pltpu.SemaphoreType.DMA,
            pltpu.SemaphoreType.DMA,
            pltpu.SemaphoreType.DMA,
            pltpu.SemaphoreType.DMA,
        ],
        compiler_params=pltpu.CompilerParams(needs_layout_passes=False),
    )
    return call(table_src, table_dst, ea3, src3, dst3, params, zeros)


def _final_body(n0_ref, n1_ref, d0_ref, d1_ref, wo_ref, bo_ref, g_ref, b_ref,
                o_ref):
    num = n0_ref[...] + n1_ref[...]
    den = d0_ref[...] + d1_ref[...]
    agg = jnp.where(den > 0.0, num / den, 0.0)
    o = jnp.dot(agg, wo_ref[...], preferred_element_type=_f32) + bo_ref[...]
    mu = jnp.mean(o, axis=-1, keepdims=True)
    var = jnp.mean((o - mu) ** 2, axis=-1, keepdims=True)
    o = g_ref[...] * (o - mu) * lax.rsqrt(var + 1e-5) + b_ref[...]
    o_ref[...] = jnp.maximum(o, 0.0)


def _final_tc(n0, n1, d0, d1, Wo, bo, g, b):
    blk = 400
    grid = N // blk
    return pl.pallas_call(
        _final_body,
        grid=(grid,),
        in_specs=[
            pl.BlockSpec((blk, H), lambda i: (i, 0)),
            pl.BlockSpec((blk, H), lambda i: (i, 0)),
            pl.BlockSpec((blk, 1), lambda i: (i, 0)),
            pl.BlockSpec((blk, 1), lambda i: (i, 0)),
            pl.BlockSpec((H, H), lambda i: (0, 0)),
            pl.BlockSpec((1, H), lambda i: (0, 0)),
            pl.BlockSpec((1, H), lambda i: (0, 0)),
            pl.BlockSpec((1, H), lambda i: (0, 0)),
        ],
        out_specs=pl.BlockSpec((blk, H), lambda i: (i, 0)),
        out_shape=jax.ShapeDtypeStruct((N, H), _f32),
    )(n0, n1, d0, d1, Wo, bo, g, b)


def kernel(node_features, edge_index, edge_attr, Wn, bn, We, be,
           Wa1, ba1, Wa2, ba2, Wo, bo, ln_g, ln_b):
    tl = jnp.float32(2.0)
    A1 = Wa1[:H] * tl
    A2 = Wa1[H:2 * H] * tl
    A3 = Wa1[2 * H:] * tl
    table_src, table_dst = _node_tc(node_features, Wn, bn.reshape(1, H), A1, A2)
    ea3 = _edge_tc(edge_attr, We, be.reshape(1, H), A3, (ba1 * tl).reshape(1, H))
    cv = jnp.sum(Wa2[:, 0]) + ba2[0]
    params = jnp.concatenate([Wa2[:, 0] * 2.0,
                              jnp.full((ACCW - H,), cv, _f32)])
    zeros = jnp.zeros((N, H), _f32)
    num, den = _sc_edge(table_src, table_dst, ea3,
                        edge_index[0], edge_index[1], params, zeros)
    d0 = den[0].reshape(DENR * H)[:N].reshape(N, 1)
    d1 = den[1].reshape(DENR * H)[:N].reshape(N, 1)
    return _final_tc(num[0], num[1], d0, d1, Wo, bo.reshape(1, H),
                     ln_g.reshape(1, H), ln_b.reshape(1, H))


# speedup vs baseline: 1.5232x; 1.2779x over previous
import jax
import jax.numpy as jnp
from jax import lax
from jax.experimental import pallas as pl
from jax.experimental.pallas import tpu as pltpu
from jax.experimental.pallas import tpu_sc as plsc

N = 10000
E = 320000
H = 128
D_EDGE = 16
ACCW = 144
NW = 32
EPW = E // NW
B = 16
CH = EPW // B
BCH = 125
NBL = CH // BCH
RPT = 624
DENR = 80

_f32 = jnp.float32


def _node_body(x_ref, wn_ref, bn_ref, a1_ref, a2_ref, o1_ref, o2_ref):
    h = jnp.dot(x_ref[...], wn_ref[...], preferred_element_type=_f32) + bn_ref[...]
    o1_ref[...] = jnp.dot(h, a1_ref[...], preferred_element_type=_f32)
    o2_ref[:, :H] = jnp.dot(h, a2_ref[...], preferred_element_type=_f32)
    o2_ref[:, H:] = h


def _node_tc(x, Wn, bn, A1, A2):
    blk = 400
    grid = N // blk
    return pl.pallas_call(
        _node_body,
        grid=(grid,),
        in_specs=[
            pl.BlockSpec((blk, H), lambda i: (i, 0)),
            pl.BlockSpec((H, H), lambda i: (0, 0)),
            pl.BlockSpec((1, H), lambda i: (0, 0)),
            pl.BlockSpec((H, H), lambda i: (0, 0)),
            pl.BlockSpec((H, H), lambda i: (0, 0)),
        ],
        out_specs=[
            pl.BlockSpec((blk, H), lambda i: (i, 0)),
            pl.BlockSpec((blk, 2 * H), lambda i: (i, 0)),
        ],
        out_shape=[
            jax.ShapeDtypeStruct((N, H), _f32),
            jax.ShapeDtypeStruct((N, 2 * H), _f32),
        ],
    )(x, Wn, bn, A1, A2)


def _edge_body(ea_ref, we_ref, be_ref, a3_ref, ba1_ref, o_ref):
    a3 = a3_ref[...]
    wc = jnp.dot(we_ref[...], a3, preferred_element_type=_f32)
    bc = jnp.dot(be_ref[...], a3, preferred_element_type=_f32) + ba1_ref[...]
    o_ref[...] = jnp.dot(ea_ref[...], wc, preferred_element_type=_f32) + bc


def _edge_tc(ea, We, be, A3, ba1):
    blk = 2000
    grid = E // blk
    return pl.pallas_call(
        _edge_body,
        grid=(grid,),
        in_specs=[
            pl.BlockSpec((blk, D_EDGE), lambda i: (i, 0)),
            pl.BlockSpec((D_EDGE, H), lambda i: (0, 0)),
            pl.BlockSpec((1, H), lambda i: (0, 0)),
            pl.BlockSpec((H, H), lambda i: (0, 0)),
            pl.BlockSpec((1, H), lambda i: (0, 0)),
        ],
        out_specs=pl.BlockSpec((blk, H), lambda i: (i, 0)),
        out_shape=jax.ShapeDtypeStruct((E, H), _f32),
    )(ea, We, be, A3, ba1)


def _sc_body(tsrc, tdst, ea3, src3, dst3, params, zeros, out_num, out_den,
             acc, den_sh, sidx1, didx1, Sv0, Sv1, Dv0, Dv1, E0, E1,
             outv0, outv1, svbuf, scat0, scat1, den_t, parv, idbuf,
             semA, semB, semS0, semS1):
    cid = lax.axis_index("c")
    sid = lax.axis_index("s")
    wid = cid * 16 + sid

    pltpu.sync_copy(params, parv)
    pltpu.sync_copy(zeros.at[pl.ds(0, DENR)], den_t)
    row0 = pl.multiple_of(sid * RPT, 8)
    tail = 16 * RPT
    pltpu.sync_copy(zeros.at[pl.ds(row0, RPT)], acc.at[pl.ds(row0, RPT)])

    @pl.when(sid == 15)
    def _zero_tail():
        pltpu.sync_copy(zeros.at[pl.ds(tail, N - tail)],
                        acc.at[pl.ds(tail, N - tail)])

    @pl.when(sid == 0)
    def _zero_den():
        pltpu.sync_copy(zeros.at[pl.ds(0, DENR)], den_sh)

    for k in range(DENR // 16):
        idbuf[pl.ds(16 * k, 16)] = lax.iota(jnp.int32, 16) + 16 * k

    w2 = [parv[pl.ds(16 * j, 16)] for j in range(8)]
    cvv = parv[pl.ds(H, 16)]
    lanes = lax.iota(jnp.int32, 16)
    perms = [(lanes + sh) & 15 for sh in (8, 4, 2, 1)]
    ilv = plsc.PackFormat.INTERLEAVED

    plsc.subcore_barrier()

    def issue(cl, Sb, Db, Eb, sem, gbase):
        o = pl.multiple_of(cl * B, 8)
        pltpu.async_copy(tsrc.at[sidx1.at[pl.ds(o, B)]], Sb, sem)
        pltpu.async_copy(tdst.at[didx1.at[pl.ds(o, B)]], Db, sem)
        pltpu.async_copy(ea3.at[pl.ds((gbase + cl) * B, B)], Eb, sem)

    def wait_buf(Sb, Db, Eb, sem):
        pltpu.make_async_copy(tsrc.at[pl.ds(0, B)], Sb, sem).wait()
        pltpu.make_async_copy(tdst.at[pl.ds(0, B)], Db, sem).wait()
        pltpu.make_async_copy(ea3.at[pl.ds(0, B)], Eb, sem).wait()

    def compute(cl, gc, Sb, Db, Eb, outb, scatb, semS):
        @pl.when(gc >= 2)
        def _wait_prev_scatter():
            pltpu.make_async_copy(outb, acc.at[pl.ds(0, B)], semS).wait()

        def edge_body(e, carry):
            a0 = jnp.zeros((16,), _f32)
            a1 = jnp.zeros((16,), _f32)
            for j in range(8):
                sl = pl.ds(16 * j, 16)
                x2 = Sb[e, sl] + Db[e, sl] + Eb[e, sl]
                r = 1.0 / (jnp.exp(x2) + 1.0)
                if j % 2 == 0:
                    a0 = a0 + w2[j] * r
                else:
                    a1 = a1 + w2[j] * r
            red = a0 + a1
            for p in perms:
                red = red + red.at[p].get(mode="promise_in_bounds")
            sv = jnp.exp(cvv - red)
            for j in range(8):
                outb[e, pl.ds(16 * j, 16)] = sv * Db[e, pl.ds(H + 16 * j, 16)]
            plsc.store_scatter(svbuf, [lanes], sv, mask=lanes == e)
            return carry

        lax.fori_loop(0, B, edge_body, 0, unroll=4)
        srcv = sidx1[pl.ds(pl.multiple_of(cl * B, 8), B)]
        scatb[...] = srcv
        sval = svbuf[...]
        plsc.addupdate_scatter(
            den_t, [lax.shift_right_logical(srcv, 7), srcv & 127], sval)
        pltpu.async_copy(outb, acc.at[scatb], semS, add=True)

    def block_fn(bi, carry):
        gbase = wid * CH + bi * BCH
        pltpu.sync_copy(src3.at[wid, bi], sidx1)
        pltpu.sync_copy(dst3.at[wid, bi], didx1)
        issue(0, Sv0, Dv0, E0, semA, gbase)

        def pair_fn(k, carry2):
            cl = k * 2
            issue(cl + 1, Sv1, Dv1, E1, semB, gbase)
            wait_buf(Sv0, Dv0, E0, semA)
            compute(cl, bi * BCH + cl, Sv0, Dv0, E0, outv0, scat0, semS0)
            issue(cl + 2, Sv0, Dv0, E0, semA, gbase)
            wait_buf(Sv1, Dv1, E1, semB)
            compute(cl + 1, bi * BCH + cl + 1, Sv1, Dv1, E1, outv1, scat1,
                    semS1)
            return carry2

        lax.fori_loop(0, (BCH - 1) // 2, pair_fn, 0)
        wait_buf(Sv0, Dv0, E0, semA)
        compute(BCH - 1, bi * BCH + BCH - 1, Sv0, Dv0, E0, outv0, scat0, semS0)
        return carry

    lax.fori_loop(0, NBL, block_fn, 0)

    pltpu.make_async_copy(outv0, acc.at[pl.ds(0, B)], semS0).wait()
    pltpu.make_async_copy(outv1, acc.at[pl.ds(0, B)], semS1).wait()

    pltpu.sync_copy(den_t, den_sh.at[idbuf], add=True)
    plsc.subcore_barrier()

    pltpu.sync_copy(acc.at[pl.ds(row0, RPT)], out_num.at[cid, pl.ds(row0, RPT)])

    @pl.when(sid == 15)
    def _write_tail():
        pltpu.sync_copy(acc.at[pl.ds(tail, N - tail)],
                        out_num.at[cid, pl.ds(tail, N - tail)])

    @pl.when(sid == 0)
    def _write_den():
        pltpu.sync_copy(den_sh, out_den.at[cid])


def _sc_edge(table_src, table_dst, ea3, src, dst, params, zeros):
    src3 = src.reshape(NW, NBL, BCH * B)
    dst3 = dst.reshape(NW, NBL, BCH * B)
    mesh = plsc.VectorSubcoreMesh(core_axis_name="c", subcore_axis_name="s")
    call = pl.kernel(
        _sc_body,
        out_type=(
            jax.ShapeDtypeStruct((2, N, H), _f32),
            jax.ShapeDtypeStruct((2, DENR, H), _f32),
        ),
        mesh=mesh,
        scratch_types=[
            pltpu.VMEM_SHARED((N, H), _f32),
            pltpu.VMEM_SHARED((DENR, H), _f32),
            pltpu.VMEM((BCH * B,), jnp.int32),
            pltpu.VMEM((BCH * B,), jnp.int32),
            pltpu.VMEM((B, H), _f32),
            pltpu.VMEM((B, H), _f32),
            pltpu.VMEM((B, 2 * H), _f32),
            pltpu.VMEM((B, 2 * H), _f32),
            pltpu.VMEM((B, H), _f32),
            pltpu.VMEM((B, H), _f32),
            pltpu.VMEM((B, H), _f32),
            pltpu.VMEM((B, H), _f32),
            pltpu.VMEM((16,), _f32),
            pltpu.VMEM((16,), jnp.int32),
            pltpu.VMEM((16,), jnp.int32),
            pltpu.VMEM((DENR, H), _f32),
            pltpu.VMEM((ACCW,), _f32),
            pltpu.VMEM((DENR,), jnp.int32),
            pltpu.SemaphoreType.DMA,
            pltpu.SemaphoreType.DMA,
            pltpu.SemaphoreType.DMA,
            pltpu.SemaphoreType.DMA,
        ],
        compiler_params=pltpu.CompilerParams(needs_layout_passes=False),
    )
    return call(table_src, table_dst, ea3, src3, dst3, params, zeros)


def _final_body(n0_ref, n1_ref, d0_ref, d1_ref, wo_ref, bo_ref, g_ref, b_ref,
                o_ref):
    num = n0_ref[...] + n1_ref[...]
    den = d0_ref[...] + d1_ref[...]
    agg = jnp.where(den > 0.0, num / den, 0.0)
    o = jnp.dot(agg, wo_ref[...], preferred_element_type=_f32) + bo_ref[...]
    mu = jnp.mean(o, axis=-1, keepdims=True)
    var = jnp.mean((o - mu) ** 2, axis=-1, keepdims=True)
    o = g_ref[...] * (o - mu) * lax.rsqrt(var + 1e-5) + b_ref[...]
    o_ref[...] = jnp.maximum(o, 0.0)


def _final_tc(n0, n1, d0, d1, Wo, bo, g, b):
    blk = 400
    grid = N // blk
    return pl.pallas_call(
        _final_body,
        grid=(grid,),
        in_specs=[
            pl.BlockSpec((blk, H), lambda i: (i, 0)),
            pl.BlockSpec((blk, H), lambda i: (i, 0)),
            pl.BlockSpec((blk, 1), lambda i: (i, 0)),
            pl.BlockSpec((blk, 1), lambda i: (i, 0)),
            pl.BlockSpec((H, H), lambda i: (0, 0)),
            pl.BlockSpec((1, H), lambda i: (0, 0)),
            pl.BlockSpec((1, H), lambda i: (0, 0)),
            pl.BlockSpec((1, H), lambda i: (0, 0)),
        ],
        out_specs=pl.BlockSpec((blk, H), lambda i: (i, 0)),
        out_shape=jax.ShapeDtypeStruct((N, H), _f32),
    )(n0, n1, d0, d1, Wo, bo, g, b)


def kernel(node_features, edge_index, edge_attr, Wn, bn, We, be,
           Wa1, ba1, Wa2, ba2, Wo, bo, ln_g, ln_b):
    tl = jnp.float32(2.0)
    A1 = Wa1[:H] * tl
    A2 = Wa1[H:2 * H] * tl
    A3 = Wa1[2 * H:] * tl
    table_src, table_dst = _node_tc(node_features, Wn, bn.reshape(1, H), A1, A2)
    ea3 = _edge_tc(edge_attr, We, be.reshape(1, H), A3, (ba1 * tl).reshape(1, H))
    cv = jnp.sum(Wa2[:, 0]) + ba2[0]
    params = jnp.concatenate([Wa2[:, 0] * 2.0,
                              jnp.full((ACCW - H,), cv, _f32)])
    zeros = jnp.zeros((N, H), _f32)
    num, den = _sc_edge(table_src, table_dst, ea3,
                        edge_index[0], edge_index[1], params, zeros)
    d0 = den[0].reshape(DENR * H)[:N].reshape(N, 1)
    d1 = den[1].reshape(DENR * H)[:N].reshape(N, 1)
    return _final_tc(num[0], num[1], d0, d1, Wo, bo.reshape(1, H),
                     ln_g.reshape(1, H), ln_b.reshape(1, H))


# speedup vs baseline: 1.9995x; 1.3127x over previous
import jax
import jax.numpy as jnp
from jax import lax
from jax.experimental import pallas as pl
from jax.experimental.pallas import tpu as pltpu
from jax.experimental.pallas import tpu_sc as plsc

N = 10000
E = 320000
H = 128
D_EDGE = 16
ACCW = 144
NW = 32
EPW = E // NW
B = 16
CH = EPW // B
BCH = 125
NBL = CH // BCH
RPT = 624
DENR = 80

_f32 = jnp.float32


def _node_body(x_ref, wn_ref, bn_ref, a1_ref, a2_ref, o1_ref, o2_ref):
    h = jnp.dot(x_ref[...], wn_ref[...], preferred_element_type=_f32) + bn_ref[...]
    o1_ref[...] = jnp.dot(h, a1_ref[...], preferred_element_type=_f32)
    o2_ref[:, :H] = jnp.dot(h, a2_ref[...], preferred_element_type=_f32)
    o2_ref[:, H:] = h


def _node_tc(x, Wn, bn, A1, A2):
    blk = 400
    grid = N // blk
    return pl.pallas_call(
        _node_body,
        grid=(grid,),
        in_specs=[
            pl.BlockSpec((blk, H), lambda i: (i, 0)),
            pl.BlockSpec((H, H), lambda i: (0, 0)),
            pl.BlockSpec((1, H), lambda i: (0, 0)),
            pl.BlockSpec((H, H), lambda i: (0, 0)),
            pl.BlockSpec((H, H), lambda i: (0, 0)),
        ],
        out_specs=[
            pl.BlockSpec((blk, H), lambda i: (i, 0)),
            pl.BlockSpec((blk, 2 * H), lambda i: (i, 0)),
        ],
        out_shape=[
            jax.ShapeDtypeStruct((N, H), _f32),
            jax.ShapeDtypeStruct((N, 2 * H), _f32),
        ],
    )(x, Wn, bn, A1, A2)


def _edge_body(ea_ref, we_ref, be_ref, a3_ref, ba1_ref, o_ref):
    a3 = a3_ref[...]
    wc = jnp.dot(we_ref[...], a3, preferred_element_type=_f32)
    bc = jnp.dot(be_ref[...], a3, preferred_element_type=_f32) + ba1_ref[...]
    o_ref[...] = jnp.dot(ea_ref[...], wc, preferred_element_type=_f32) + bc


def _edge_tc(ea, We, be, A3, ba1):
    blk = 2000
    grid = E // blk
    return pl.pallas_call(
        _edge_body,
        grid=(grid,),
        in_specs=[
            pl.BlockSpec((blk, D_EDGE), lambda i: (i, 0)),
            pl.BlockSpec((D_EDGE, H), lambda i: (0, 0)),
            pl.BlockSpec((1, H), lambda i: (0, 0)),
            pl.BlockSpec((H, H), lambda i: (0, 0)),
            pl.BlockSpec((1, H), lambda i: (0, 0)),
        ],
        out_specs=pl.BlockSpec((blk, H), lambda i: (i, 0)),
        out_shape=jax.ShapeDtypeStruct((E, H), _f32),
    )(ea, We, be, A3, ba1)


def _sc_body(tsrc, tdst, ea3, src3, dst3, params, zeros, out_num, out_den,
             acc, den_sh, sidx1, didx1, Sv0, Sv1, Dv0, Dv1, E0, E1,
             outv0, outv1, svbuf, scat0, scat1, den_t, parv, idbuf,
             semA, semB, semS0, semS1):
    cid = lax.axis_index("c")
    sid = lax.axis_index("s")
    wid = cid * 16 + sid

    pltpu.sync_copy(params, parv)
    pltpu.sync_copy(zeros.at[pl.ds(0, DENR)], den_t)
    row0 = pl.multiple_of(sid * RPT, 8)
    tail = 16 * RPT
    pltpu.sync_copy(zeros.at[pl.ds(row0, RPT)], acc.at[pl.ds(row0, RPT)])

    @pl.when(sid == 15)
    def _zero_tail():
        pltpu.sync_copy(zeros.at[pl.ds(tail, N - tail)],
                        acc.at[pl.ds(tail, N - tail)])

    @pl.when(sid == 0)
    def _zero_den():
        pltpu.sync_copy(zeros.at[pl.ds(0, DENR)], den_sh)

    for k in range(DENR // 16):
        idbuf[pl.ds(16 * k, 16)] = lax.iota(jnp.int32, 16) + 16 * k

    w2 = [parv[pl.ds(16 * j, 16)] for j in range(8)]
    cvv = parv[pl.ds(H, 16)]
    lanes = lax.iota(jnp.int32, 16)
    perms = [(lanes + sh) & 15 for sh in (8, 4, 2, 1)]
    ilv = plsc.PackFormat.INTERLEAVED

    plsc.subcore_barrier()

    def issue(cl, Sb, Db, Eb, sem, gbase):
        o = pl.multiple_of(cl * B, 8)
        pltpu.async_copy(tsrc.at[sidx1.at[pl.ds(o, B)]], Sb, sem)
        pltpu.async_copy(tdst.at[didx1.at[pl.ds(o, B)]], Db, sem)
        pltpu.async_copy(ea3.at[pl.ds((gbase + cl) * B, B)], Eb, sem)

    def wait_buf(Sb, Db, Eb, sem):
        pltpu.make_async_copy(tsrc.at[pl.ds(0, B)], Sb, sem).wait()
        pltpu.make_async_copy(tdst.at[pl.ds(0, B)], Db, sem).wait()
        pltpu.make_async_copy(ea3.at[pl.ds(0, B)], Eb, sem).wait()

    def compute(cl, gc, Sb, Db, Eb, outb, scatb, semS):
        @pl.when(gc >= 2)
        def _wait_prev_scatter():
            pltpu.make_async_copy(outb, acc.at[pl.ds(0, B)], semS).wait()

        def edge_body(e, carry):
            a0 = jnp.zeros((16,), _f32)
            a1 = jnp.zeros((16,), _f32)
            for j in range(8):
                sl = pl.ds(16 * j, 16)
                x2 = Sb[e, sl] + Db[e, sl] + Eb[e, sl]
                r = 1.0 / (jnp.exp(x2) + 1.0)
                if j % 2 == 0:
                    a0 = a0 + w2[j] * r
                else:
                    a1 = a1 + w2[j] * r
            red = a0 + a1
            for p in perms:
                red = red + red.at[p].get(mode="promise_in_bounds")
            sv = jnp.exp(cvv - red)
            for j in range(8):
                outb[e, pl.ds(16 * j, 16)] = sv * Db[e, pl.ds(H + 16 * j, 16)]
            plsc.store_scatter(svbuf, [lanes], sv, mask=lanes == e)
            return carry

        lax.fori_loop(0, B, edge_body, 0, unroll=2)
        srcv = sidx1[pl.ds(pl.multiple_of(cl * B, 8), B)]
        scatb[...] = srcv
        sval = svbuf[...]
        plsc.addupdate_scatter(
            den_t, [lax.shift_right_logical(srcv, 7), srcv & 127], sval)
        pltpu.async_copy(outb, acc.at[scatb], semS, add=True)

    def block_fn(bi, carry):
        gbase = wid * CH + bi * BCH
        pltpu.sync_copy(src3.at[wid, bi], sidx1)
        pltpu.sync_copy(dst3.at[wid, bi], didx1)
        issue(0, Sv0, Dv0, E0, semA, gbase)

        def pair_fn(k, carry2):
            cl = k * 2
            issue(cl + 1, Sv1, Dv1, E1, semB, gbase)
            wait_buf(Sv0, Dv0, E0, semA)
            compute(cl, bi * BCH + cl, Sv0, Dv0, E0, outv0, scat0, semS0)
            issue(cl + 2, Sv0, Dv0, E0, semA, gbase)
            wait_buf(Sv1, Dv1, E1, semB)
            compute(cl + 1, bi * BCH + cl + 1, Sv1, Dv1, E1, outv1, scat1,
                    semS1)
            return carry2

        lax.fori_loop(0, (BCH - 1) // 2, pair_fn, 0)
        wait_buf(Sv0, Dv0, E0, semA)
        compute(BCH - 1, bi * BCH + BCH - 1, Sv0, Dv0, E0, outv0, scat0, semS0)
        return carry

    lax.fori_loop(0, NBL, block_fn, 0)

    pltpu.make_async_copy(outv0, acc.at[pl.ds(0, B)], semS0).wait()
    pltpu.make_async_copy(outv1, acc.at[pl.ds(0, B)], semS1).wait()

    pltpu.sync_copy(den_t, den_sh.at[idbuf], add=True)
    plsc.subcore_barrier()

    pltpu.sync_copy(acc.at[pl.ds(row0, RPT)], out_num.at[cid, pl.ds(row0, RPT)])

    @pl.when(sid == 15)
    def _write_tail():
        pltpu.sync_copy(acc.at[pl.ds(tail, N - tail)],
                        out_num.at[cid, pl.ds(tail, N - tail)])

    @pl.when(sid == 0)
    def _write_den():
        pltpu.sync_copy(den_sh, out_den.at[cid])


def _sc_edge(table_src, table_dst, ea3, src, dst, params, zeros):
    src3 = src.reshape(NW, NBL, BCH * B)
    dst3 = dst.reshape(NW, NBL, BCH * B)
    mesh = plsc.VectorSubcoreMesh(core_axis_name="c", subcore_axis_name="s")
    call = pl.kernel(
        _sc_body,
        out_type=(
            jax.ShapeDtypeStruct((2, N, H), _f32),
            jax.ShapeDtypeStruct((2, DENR, H), _f32),
        ),
        mesh=mesh,
        scratch_types=[
            pltpu.VMEM_SHARED((N, H), _f32),
            pltpu.VMEM_SHARED((DENR, H), _f32),
            pltpu.VMEM((BCH * B,), jnp.int32),
            pltpu.VMEM((BCH * B,), jnp.int32),
            pltpu.VMEM((B, H), _f32),
            pltpu.VMEM((B, H), _f32),
            pltpu.VMEM((B, 2 * H), _f32),
            pltpu.VMEM((B, 2 * H), _f32),
            pltpu.VMEM((B, H), _f32),
            pltpu.VMEM((B, H), _f32),
            pltpu.VMEM((B, H), _f32),
            pltpu.VMEM((B, H), _f32),
            pltpu.VMEM((16,), _f32),
            pltpu.VMEM((16,), jnp.int32),
            pltpu.VMEM((16,), jnp.int32),
            pltpu.VMEM((DENR, H), _f32),
            pltpu.VMEM((ACCW,), _f32),
            pltpu.VMEM((DENR,), jnp.int32),
            pltpu.SemaphoreType.DMA,
            pltpu.SemaphoreType.DMA,
            pltpu.SemaphoreType.DMA,
            pltpu.SemaphoreType.DMA,
        ],
        compiler_params=pltpu.CompilerParams(needs_layout_passes=False),
    )
    return call(table_src, table_dst, ea3, src3, dst3, params, zeros)


def _final_body(n0_ref, n1_ref, d0_ref, d1_ref, wo_ref, bo_ref, g_ref, b_ref,
                o_ref):
    num = n0_ref[...] + n1_ref[...]
    den = d0_ref[...] + d1_ref[...]
    agg = jnp.where(den > 0.0, num / den, 0.0)
    o = jnp.dot(agg, wo_ref[...], preferred_element_type=_f32) + bo_ref[...]
    mu = jnp.mean(o, axis=-1, keepdims=True)
    var = jnp.mean((o - mu) ** 2, axis=-1, keepdims=True)
    o = g_ref[...] * (o - mu) * lax.rsqrt(var + 1e-5) + b_ref[...]
    o_ref[...] = jnp.maximum(o, 0.0)


def _final_tc(n0, n1, d0, d1, Wo, bo, g, b):
    blk = 400
    grid = N // blk
    return pl.pallas_call(
        _final_body,
        grid=(grid,),
        in_specs=[
            pl.BlockSpec((blk, H), lambda i: (i, 0)),
            pl.BlockSpec((blk, H), lambda i: (i, 0)),
            pl.BlockSpec((blk, 1), lambda i: (i, 0)),
            pl.BlockSpec((blk, 1), lambda i: (i, 0)),
            pl.BlockSpec((H, H), lambda i: (0, 0)),
            pl.BlockSpec((1, H), lambda i: (0, 0)),
            pl.BlockSpec((1, H), lambda i: (0, 0)),
            pl.BlockSpec((1, H), lambda i: (0, 0)),
        ],
        out_specs=pl.BlockSpec((blk, H), lambda i: (i, 0)),
        out_shape=jax.ShapeDtypeStruct((N, H), _f32),
    )(n0, n1, d0, d1, Wo, bo, g, b)


def kernel(node_features, edge_index, edge_attr, Wn, bn, We, be,
           Wa1, ba1, Wa2, ba2, Wo, bo, ln_g, ln_b):
    tl = jnp.float32(2.0)
    A1 = Wa1[:H] * tl
    A2 = Wa1[H:2 * H] * tl
    A3 = Wa1[2 * H:] * tl
    table_src, table_dst = _node_tc(node_features, Wn, bn.reshape(1, H), A1, A2)
    ea3 = _edge_tc(edge_attr, We, be.reshape(1, H), A3, (ba1 * tl).reshape(1, H))
    cv = jnp.sum(Wa2[:, 0]) + ba2[0]
    params = jnp.concatenate([Wa2[:, 0] * 2.0,
                              jnp.full((ACCW - H,), cv, _f32)])
    zeros = jnp.zeros((N, H), _f32)
    num, den = _sc_edge(table_src, table_dst, ea3,
                        edge_index[0], edge_index[1], params, zeros)
    d0 = den[0].reshape(DENR * H)[:N].reshape(N, 1)
    d1 = den[1].reshape(DENR * H)[:N].reshape(N, 1)
    return _final_tc(num[0], num[1], d0, d1, Wo, bo.reshape(1, H),
                     ln_g.reshape(1, H), ln_b.reshape(1, H))
